# baseline (device time: 3783371 ns/iter reference)
import jax
import jax.numpy as jnp
from jax import lax
from jax.experimental import pallas as pl
from jax.experimental.pallas import tpu as pltpu


def _my_coords():
    return lax.axis_index("x"), lax.axis_index("y"), lax.axis_index("z")


def _gather(xb, a2d):
    T, D = xb.shape
    Ta = a2d.shape[0]

    def body(x_ref, a_ref, xs_ref, aall_ref, cp_sem_x, cp_sem_a,
             sx_send, sx_recv, sa_send, sa_recv):
        my_x, my_y, my_z = _my_coords()
        peer = (1 - my_x, my_y, my_z)

        cp_x = pltpu.make_async_copy(
            x_ref, xs_ref.at[pl.ds(my_x * T, T)], cp_sem_x
        )
        cp_a = pltpu.make_async_copy(
            a_ref, aall_ref.at[pl.ds(my_x * Ta, Ta)], cp_sem_a
        )
        cp_x.start()
        cp_a.start()

        rdma_x = pltpu.make_async_remote_copy(
            src_ref=x_ref,
            dst_ref=xs_ref.at[pl.ds(my_x * T, T)],
            send_sem=sx_send,
            recv_sem=sx_recv,
            device_id=peer,
            device_id_type=pl.DeviceIdType.MESH,
        )
        rdma_a = pltpu.make_async_remote_copy(
            src_ref=a_ref,
            dst_ref=aall_ref.at[pl.ds(my_x * Ta, Ta)],
            send_sem=sa_send,
            recv_sem=sa_recv,
            device_id=peer,
            device_id_type=pl.DeviceIdType.MESH,
        )
        rdma_x.start()
        rdma_a.start()
        cp_x.wait()
        cp_a.wait()
        rdma_x.wait()
        rdma_a.wait()

    return pl.pallas_call(
        body,
        out_shape=(
            jax.ShapeDtypeStruct((2 * T, D), xb.dtype),
            jax.ShapeDtypeStruct((2 * Ta, 128), a2d.dtype),
        ),
        in_specs=[
            pl.BlockSpec(memory_space=pltpu.MemorySpace.HBM),
            pl.BlockSpec(memory_space=pltpu.MemorySpace.HBM),
        ],
        out_specs=(
            pl.BlockSpec(memory_space=pltpu.MemorySpace.HBM),
            pl.BlockSpec(memory_space=pltpu.MemorySpace.HBM),
        ),
        scratch_shapes=[
            pltpu.SemaphoreType.DMA,
            pltpu.SemaphoreType.DMA,
            pltpu.SemaphoreType.DMA,
            pltpu.SemaphoreType.DMA,
            pltpu.SemaphoreType.DMA,
            pltpu.SemaphoreType.DMA,
        ],
    )(xb, a2d)


_CAP = 1536
_TM = 512


def _moe_routed(xp, W1b, W2b):
    R, D = xp.shape
    E, _, F = W1b.shape
    FB = 1024
    NB, NF = R // _TM, F // FB
    BPE = _CAP // _TM

    def body(x_ref, w1_ref, w2_ref, out_ref, acc_ref):
        f = pl.program_id(1)
        h = jnp.maximum(
            jnp.dot(x_ref[...], w1_ref[0], preferred_element_type=jnp.float32),
            0.0,
        ).astype(jnp.bfloat16)
        prod = jnp.dot(h, w2_ref[0], preferred_element_type=jnp.float32)

        @pl.when(f == 0)
        def _():
            acc_ref[...] = prod

        @pl.when(f != 0)
        def _():
            acc_ref[...] += prod

        @pl.when(f == NF - 1)
        def _():
            out_ref[...] = acc_ref[...].astype(jnp.bfloat16)

    return pl.pallas_call(
        body,
        grid=(NB, NF),
        in_specs=[
            pl.BlockSpec((_TM, D), lambda b, f: (b, 0)),
            pl.BlockSpec((1, D, FB), lambda b, f: (b // BPE, 0, f)),
            pl.BlockSpec((1, FB, D), lambda b, f: (b // BPE, f, 0)),
        ],
        out_specs=pl.BlockSpec((_TM, D), lambda b, f: (b, 0)),
        out_shape=jax.ShapeDtypeStruct((R, D), jnp.bfloat16),
        scratch_shapes=[pltpu.VMEM((_TM, D), jnp.float32)],
        compiler_params=pltpu.CompilerParams(
            dimension_semantics=("parallel", "arbitrary"),
        ),
    )(xp, W1b, W2b)


def _exchange_partial(partial, T):
    TT, D = partial.shape

    def body(p_ref, out_ref, send_sem, recv_sem):
        my_x, my_y, my_z = _my_coords()
        peer = (1 - my_x, my_y, my_z)
        rdma = pltpu.make_async_remote_copy(
            src_ref=p_ref.at[pl.ds((1 - my_x) * T, T)],
            dst_ref=out_ref,
            send_sem=send_sem,
            recv_sem=recv_sem,
            device_id=peer,
            device_id_type=pl.DeviceIdType.MESH,
        )
        rdma.start()
        rdma.wait()

    return pl.pallas_call(
        body,
        out_shape=jax.ShapeDtypeStruct((T, D), partial.dtype),
        in_specs=[pl.BlockSpec(memory_space=pltpu.MemorySpace.HBM)],
        out_specs=pl.BlockSpec(memory_space=pltpu.MemorySpace.HBM),
        scratch_shapes=[pltpu.SemaphoreType.DMA, pltpu.SemaphoreType.DMA],
    )(partial)


def kernel(x, assign, W1, W2):
    T, D = x.shape
    E = W1.shape[0]

    xb = x.astype(jnp.bfloat16)
    W1b = W1.astype(jnp.bfloat16)
    W2b = W2.astype(jnp.bfloat16)
    a2d = assign.reshape(T // 128, 128)

    xs, a_all = _gather(xb, a2d)
    a = a_all.reshape(2 * T)

    my_x = lax.axis_index("x")
    le = a - my_x * E
    is_mine = jnp.logical_and(le >= 0, le < E)
    lec = jnp.clip(le, 0, E - 1)
    onehot = jnp.logical_and(
        lec[:, None] == jnp.arange(E)[None, :], is_mine[:, None]
    ).astype(jnp.int32)
    excl_rank = jnp.cumsum(onehot, axis=0) - onehot
    rank = jnp.take_along_axis(excl_rank, lec[:, None], axis=1)[:, 0]
    ok = jnp.logical_and(is_mine, rank < _CAP)
    dest = jnp.where(ok, lec * _CAP + rank, E * _CAP)

    src_idx = (
        jnp.full((E * _CAP,), 2 * T, jnp.int32)
        .at[dest]
        .set(jnp.arange(2 * T, dtype=jnp.int32), mode="drop")
    )
    xp = jnp.take(xs, src_idx, axis=0, mode="fill", fill_value=0)
    part_sorted = _moe_routed(xp, W1b, W2b)

    gidx = jnp.where(ok, dest, E * _CAP)
    partial = jnp.take(part_sorted, gidx, axis=0, mode="fill", fill_value=0)

    peer_part = _exchange_partial(partial, T)

    mine = lax.dynamic_slice_in_dim(partial, my_x * T, T, axis=0)
    return (mine.astype(jnp.float32) + peer_part.astype(jnp.float32))


# device time: 1158709 ns/iter; 3.2652x vs baseline; 3.2652x over previous
import jax
import jax.numpy as jnp
from jax import lax
from jax.experimental import pallas as pl
from jax.experimental.pallas import tpu as pltpu

_HBM = pltpu.MemorySpace.HBM


def _my_coords():
    return lax.axis_index("x"), lax.axis_index("y"), lax.axis_index("z")


def _slice_gather(xb, a2d):
    T, D = xb.shape
    TH = T // 2
    AR = a2d.shape[0] // 2

    def body(x_ref, a_ref, xs_ref, as_ref, cpx, cpa, sx_s, sx_r, sa_s, sa_r):
        mx, my, mz = _my_coords()
        rows = pl.ds(mz * TH, TH)
        arows = pl.ds(mz * AR, AR)
        is_src = mx == my

        rdma_x = pltpu.make_async_remote_copy(
            src_ref=x_ref.at[rows],
            dst_ref=xs_ref,
            send_sem=sx_s,
            recv_sem=sx_r,
            device_id=(1 - mx, my, mz),
            device_id_type=pl.DeviceIdType.MESH,
        )
        rdma_a = pltpu.make_async_remote_copy(
            src_ref=a_ref.at[arows],
            dst_ref=as_ref,
            send_sem=sa_s,
            recv_sem=sa_r,
            device_id=(1 - mx, my, mz),
            device_id_type=pl.DeviceIdType.MESH,
        )

        @pl.when(is_src)
        def _():
            cp1 = pltpu.make_async_copy(x_ref.at[rows], xs_ref, cpx)
            cp2 = pltpu.make_async_copy(a_ref.at[arows], as_ref, cpa)
            cp1.start()
            cp2.start()
            rdma_x.start()
            rdma_a.start()
            cp1.wait()
            cp2.wait()
            rdma_x.wait_send()
            rdma_a.wait_send()

        @pl.when(jnp.logical_not(is_src))
        def _():
            rdma_x.wait_recv()
            rdma_a.wait_recv()

    return pl.pallas_call(
        body,
        out_shape=(
            jax.ShapeDtypeStruct((TH, D), xb.dtype),
            jax.ShapeDtypeStruct((AR, 128), a2d.dtype),
        ),
        in_specs=[
            pl.BlockSpec(memory_space=_HBM),
            pl.BlockSpec(memory_space=_HBM),
        ],
        out_specs=(
            pl.BlockSpec(memory_space=_HBM),
            pl.BlockSpec(memory_space=_HBM),
        ),
        scratch_shapes=[
            pltpu.SemaphoreType.DMA,
            pltpu.SemaphoreType.DMA,
            pltpu.SemaphoreType.DMA,
            pltpu.SemaphoreType.DMA,
            pltpu.SemaphoreType.DMA,
            pltpu.SemaphoreType.DMA,
        ],
    )(xb, a2d)


def _moe_slice(xsl, a_bcast, W1b, W2b):
    TH, D = xsl.shape
    E, _, F = W1b.shape
    TM = 512
    FB = 1024
    NT, NF = TH // TM, F // FB

    def body(x_ref, a_ref, w1_ref, w2_ref, out_ref, acc_ref):
        e = pl.program_id(1)
        f = pl.program_id(2)
        mx = lax.axis_index("x")
        e_gid = mx * E + e

        mask = a_ref[:, 0:1] == e_gid
        xm = jnp.where(mask, x_ref[...], jnp.zeros((), xsl.dtype))
        h = jnp.maximum(
            jnp.dot(xm, w1_ref[0], preferred_element_type=jnp.float32), 0.0
        ).astype(jnp.bfloat16)
        prod = jnp.dot(h, w2_ref[0], preferred_element_type=jnp.float32)

        first = jnp.logical_and(e == 0, f == 0)

        @pl.when(first)
        def _():
            acc_ref[...] = prod

        @pl.when(jnp.logical_not(first))
        def _():
            acc_ref[...] += prod

        @pl.when(jnp.logical_and(e == E - 1, f == NF - 1))
        def _():
            out_ref[...] = acc_ref[...].astype(jnp.bfloat16)

    return pl.pallas_call(
        body,
        grid=(NT, E, NF),
        in_specs=[
            pl.BlockSpec((TM, D), lambda t, e, f: (t, 0)),
            pl.BlockSpec((TM, 128), lambda t, e, f: (t, 0)),
            pl.BlockSpec((1, D, FB), lambda t, e, f: (e, 0, f)),
            pl.BlockSpec((1, FB, D), lambda t, e, f: (e, f, 0)),
        ],
        out_specs=pl.BlockSpec((TM, D), lambda t, e, f: (t, 0)),
        out_shape=jax.ShapeDtypeStruct((TH, D), jnp.bfloat16),
        scratch_shapes=[pltpu.VMEM((TM, D), jnp.float32)],
        compiler_params=pltpu.CompilerParams(
            dimension_semantics=("parallel", "arbitrary", "arbitrary"),
        ),
    )(xsl, a_bcast, W1b, W2b)


def _scatter_combine(pslice):
    TH, D = pslice.shape

    def body(p_ref, slabs_ref, cp_sem, send_sems, recv_sems):
        mx, my, mz = _my_coords()
        slot = mz * 2 + mx

        for yr in range(2):
            for zr in range(2):
                idx = yr * 2 + zr
                is_self = jnp.logical_and(
                    jnp.logical_and(my == mx, yr == my), zr == mz
                )
                rdma = pltpu.make_async_remote_copy(
                    src_ref=p_ref,
                    dst_ref=slabs_ref.at[slot],
                    send_sem=send_sems.at[idx],
                    recv_sem=recv_sems.at[slot],
                    device_id=(my, yr, zr),
                    device_id_type=pl.DeviceIdType.MESH,
                )

                @pl.when(is_self)
                def _(rdma=rdma):
                    cp = pltpu.make_async_copy(p_ref, slabs_ref.at[slot], cp_sem)
                    cp.start()
                    cp.wait()

                @pl.when(jnp.logical_not(is_self))
                def _(rdma=rdma):
                    rdma.start()
                    rdma.wait_send()

        for c in range(2):
            for ep in range(2):
                k = c * 2 + ep
                from_self = jnp.logical_and(
                    jnp.logical_and(ep == mx, mx == my), c == mz
                )
                rdma = pltpu.make_async_remote_copy(
                    src_ref=p_ref,
                    dst_ref=slabs_ref.at[k],
                    send_sem=send_sems.at[0],
                    recv_sem=recv_sems.at[k],
                    device_id=(mx, my, mz),
                    device_id_type=pl.DeviceIdType.MESH,
                )

                @pl.when(jnp.logical_not(from_self))
                def _(rdma=rdma):
                    rdma.wait_recv()

    return pl.pallas_call(
        body,
        out_shape=jax.ShapeDtypeStruct((4, TH, D), pslice.dtype),
        in_specs=[pl.BlockSpec(memory_space=_HBM)],
        out_specs=pl.BlockSpec(memory_space=_HBM),
        scratch_shapes=[
            pltpu.SemaphoreType.DMA,
            pltpu.SemaphoreType.DMA((4,)),
            pltpu.SemaphoreType.DMA((4,)),
        ],
    )(pslice)


def kernel(x, assign, W1, W2):
    T, D = x.shape
    TH = T // 2

    xb = x.astype(jnp.bfloat16)
    W1b = W1.astype(jnp.bfloat16)
    W2b = W2.astype(jnp.bfloat16)
    a2d = assign.reshape(T // 128, 128)

    xsl, asl = _slice_gather(xb, a2d)
    a_bcast = jnp.broadcast_to(asl.reshape(TH, 1), (TH, 128))

    pslice = _moe_slice(xsl, a_bcast, W1b, W2b)

    slabs = _scatter_combine(pslice)

    half0 = slabs[0].astype(jnp.float32) + slabs[1].astype(jnp.float32)
    half1 = slabs[2].astype(jnp.float32) + slabs[3].astype(jnp.float32)
    return jnp.concatenate([half0, half1], axis=0)
